# Initial kernel scaffold; baseline (speedup 1.0000x reference)
import jax
import jax.numpy as jnp
from jax.experimental import pallas as pl


def kernel(x, edge_index, batch, W0, b0, W1, b1, Wc, bc):
    raise NotImplementedError("write your pallas kernel here")


# speedup vs baseline: 6.3296x; 6.3296x over previous
import functools

import jax
import jax.numpy as jnp
from jax import lax
from jax.experimental import pallas as pl
from jax.experimental.pallas import tpu as pltpu
from jax.experimental.pallas import tpu_sc as plsc

_N = 10000
_E = 320000
_F = 128
_G = 64
_CLS = 39
_EPS = 1e-6
_MAXN = 1000.0

_NC = 2
_NS = 16
_EPT = _E // (_NC * _NS)
_CH = 128
_NFULL = _EPT // _CH
_TAIL = _EPT - _NFULL * _CH
_RPT = 624
_RREM = _N - _NS * _RPT


def _sinh(a):
    return 0.5 * (jnp.exp(a) - jnp.exp(-a))


def _logmap_tail(head, tail, sqrt_c):
    z = jnp.clip(head / sqrt_c + _EPS, 1.0, None)
    dist = sqrt_c * jnp.log(z + jnp.sqrt(jnp.clip(z * z - 1.0, 1e-12, None)))
    tmp = jnp.sqrt(jnp.clip(jnp.sum(tail * tail, axis=1, keepdims=True),
                            1e-12, None) + _EPS)
    return (dist / tmp) * tail


def _expmap_norm(tail, sqrt_c):
    sq = jnp.sum(tail * tail, axis=1, keepdims=True)
    lnorm = jnp.sqrt(jnp.clip(sq + _EPS, 1e-6, None))
    cut = jnp.minimum(lnorm, 50.0)
    coef = sqrt_c * _sinh(cut / sqrt_c) / lnorm
    t2 = coef * tail
    n2 = jnp.sum(t2 * t2, axis=1, keepdims=True)
    norm = jnp.sqrt(jnp.clip(n2, 1e-12, None))
    scale = jnp.minimum(1.0, _MAXN / norm)
    t2 = t2 * scale
    head = jnp.sqrt(sqrt_c * sqrt_c
                    + jnp.sum(t2 * t2, axis=1, keepdims=True))
    return head, t2


def _xtan_body(x_ref, o_ref):
    t = x_ref[...]
    n2 = jnp.sum(t * t, axis=1, keepdims=True)
    norm = jnp.sqrt(jnp.clip(n2, 1e-12, None))
    t = t * jnp.minimum(1.0, _MAXN / norm)
    n2s = jnp.sum(t * t, axis=1, keepdims=True)
    head = jnp.sqrt(1.0 + n2s)
    z = jnp.clip(head + _EPS, 1.0, None)
    dist = jnp.log(z + jnp.sqrt(jnp.clip(z * z - 1.0, 1e-12, None)))
    tmp = jnp.sqrt(jnp.clip(n2s, 1e-12, None) + _EPS)
    o_ref[...] = (dist / tmp) * t


def _xtan(xt):
    b = 2000
    return pl.pallas_call(
        _xtan_body,
        grid=(_N // b,),
        in_specs=[pl.BlockSpec((b, _F), lambda i: (i, 0))],
        out_specs=pl.BlockSpec((b, _F), lambda i: (i, 0)),
        out_shape=jax.ShapeDtypeStruct((_N, _F), jnp.float32),
    )(xt)


def _sc_body(xtan_hbm, src_hbm, dst_hbm, zero_hbm, out_hbm,
             sidx, didx, rows, sidx_t, didx_t, rows_t, acc, sem):
    c = lax.axis_index("c")
    s = lax.axis_index("s")
    tid = c * _NS + s

    rowbase = s * _RPT
    pltpu.sync_copy(zero_hbm, acc.at[pl.ds(rowbase, _RPT)])

    @pl.when(s == _NS - 1)
    def _():
        pltpu.sync_copy(zero_hbm.at[pl.ds(0, _RREM)],
                        acc.at[pl.ds(_NS * _RPT, _RREM)])

    plsc.subcore_barrier()

    ebase = tid * _EPT

    def chunk(j, carry):
        start = ebase + j * _CH
        pltpu.sync_copy(src_hbm.at[pl.ds(start, _CH)], sidx)
        pltpu.sync_copy(dst_hbm.at[pl.ds(start, _CH)], didx)
        pltpu.async_copy(xtan_hbm.at[sidx], rows, sem).wait()
        pltpu.sync_copy(rows, acc.at[didx], add=True)
        return carry

    lax.fori_loop(0, _NFULL, chunk, 0)

    start = ebase + _NFULL * _CH
    pltpu.sync_copy(src_hbm.at[pl.ds(start, _TAIL)], sidx_t)
    pltpu.sync_copy(dst_hbm.at[pl.ds(start, _TAIL)], didx_t)
    pltpu.async_copy(xtan_hbm.at[sidx_t], rows_t, sem).wait()
    pltpu.sync_copy(rows_t, acc.at[didx_t], add=True)

    plsc.subcore_barrier()

    pltpu.sync_copy(acc.at[pl.ds(rowbase, _RPT)],
                    out_hbm.at[pl.ds(c * _N + rowbase, _RPT)])

    @pl.when(s == _NS - 1)
    def _():
        pltpu.sync_copy(acc.at[pl.ds(_NS * _RPT, _RREM)],
                        out_hbm.at[pl.ds(c * _N + _NS * _RPT, _RREM)])


def _sc_scatter(x_tan, src, dst, zero_rows):
    mesh = plsc.VectorSubcoreMesh(core_axis_name="c", subcore_axis_name="s")
    f = pl.kernel(
        _sc_body,
        out_type=jax.ShapeDtypeStruct((_NC * _N, _F), jnp.float32),
        mesh=mesh,
        scratch_types=[
            pltpu.VMEM((_CH,), jnp.int32),
            pltpu.VMEM((_CH,), jnp.int32),
            pltpu.VMEM((_CH, _F), jnp.float32),
            pltpu.VMEM((_TAIL,), jnp.int32),
            pltpu.VMEM((_TAIL,), jnp.int32),
            pltpu.VMEM((_TAIL, _F), jnp.float32),
            pltpu.VMEM_SHARED((_N, _F), jnp.float32),
            pltpu.SemaphoreType.DMA,
        ],
    )
    return f(x_tan, src, dst, zero_rows)


_B2 = 2000
_NB2 = _N // _B2


def _tail_body(xt_ref, p0_ref, p1_ref, b_ref,
               w0_ref, b0_ref, w1_ref, b1_ref, wc_ref, bc_ref,
               olog_ref, oprob_ref, acc_ref):
    i = pl.program_id(0)

    htan = xt_ref[...] + p0_ref[...] + p1_ref[...]
    head, tail = _expmap_norm(htan, 1.0)

    tt = _logmap_tail(head, tail, 2.0)
    mx = jnp.dot(tt, w0_ref[...], preferred_element_type=jnp.float32) \
        + b0_ref[...]
    head, tail = _expmap_norm(mx, 2.0)
    tt = jax.nn.relu(_logmap_tail(head, tail, 2.0))
    head, tail = _expmap_norm(tt, 2.0)
    tt = _logmap_tail(head, tail, 2.0)
    mx = jnp.dot(tt, w1_ref[...], preferred_element_type=jnp.float32) \
        + b1_ref[...]
    head, tail = _expmap_norm(mx, 2.0)
    tt = jax.nn.relu(_logmap_tail(head, tail, 2.0))
    head, tail = _expmap_norm(tt, 1.0)
    tt = _logmap_tail(head, tail, 1.0)

    bvals = b_ref[...].reshape(1, _B2)
    gid = lax.broadcasted_iota(jnp.int32, (_G, _B2), 0)
    oh = jnp.where(gid == bvals, 1.0, 0.0)
    pp = jnp.dot(oh, tt, preferred_element_type=jnp.float32)

    @pl.when(i == 0)
    def _():
        acc_ref[...] = pp

    @pl.when(i > 0)
    def _():
        acc_ref[...] = acc_ref[...] + pp

    @pl.when(i == _NB2 - 1)
    def _():
        hp = acc_ref[...]
        head, tail = _expmap_norm(hp, 1.0)
        tt = _logmap_tail(head, tail, 1.0)
        mx = jnp.dot(tt, wc_ref[...], preferred_element_type=jnp.float32) \
            + bc_ref[...]
        head, tail = _expmap_norm(mx, 1.0)
        lt = _logmap_tail(head, tail, 1.0)
        olog_ref[...] = lt
        col = lax.broadcasted_iota(jnp.int32, (_G, _F), 1)
        valid = col < _CLS
        m = jnp.maximum(
            jnp.max(jnp.where(valid, lt, -1e30), axis=1, keepdims=True), 0.0)
        e = jnp.where(valid, jnp.exp(lt - m), 0.0)
        denom = jnp.sum(e, axis=1, keepdims=True) + jnp.exp(-m)
        st = e / denom
        _, tail2 = _expmap_norm(st, 1.0)
        oprob_ref[...] = tail2


def _tail_call(x_tan, p0, p1, batch3, w0p, b0p, w1p, b1p, wcp, bcp):
    blk = lambda i: (i, 0)
    fixed = lambda i: (0, 0)
    return pl.pallas_call(
        _tail_body,
        grid=(_NB2,),
        in_specs=[
            pl.BlockSpec((_B2, _F), blk),
            pl.BlockSpec((_B2, _F), blk),
            pl.BlockSpec((_B2, _F), blk),
            pl.BlockSpec((1, 1, _B2), lambda i: (i, 0, 0)),
            pl.BlockSpec((_F, _F), fixed),
            pl.BlockSpec((1, _F), fixed),
            pl.BlockSpec((_F, _F), fixed),
            pl.BlockSpec((1, _F), fixed),
            pl.BlockSpec((_F, _F), fixed),
            pl.BlockSpec((1, _F), fixed),
        ],
        out_specs=[
            pl.BlockSpec((_G, _F), fixed),
            pl.BlockSpec((_G, _F), fixed),
        ],
        out_shape=[
            jax.ShapeDtypeStruct((_G, _F), jnp.float32),
            jax.ShapeDtypeStruct((_G, _F), jnp.float32),
        ],
        scratch_shapes=[pltpu.VMEM((_G, _F), jnp.float32)],
    )(x_tan, p0, p1, batch3, w0p, b0p, w1p, b1p, wcp, bcp)


def kernel(x, edge_index, batch, W0, b0, W1, b1, Wc, bc):
    xt = x[:, 1:]
    x_tan = _xtan(xt)

    src = edge_index[0]
    dst = edge_index[1]
    zero_rows = jnp.zeros((_RPT, _F), jnp.float32)
    parts = _sc_scatter(x_tan, src, dst, zero_rows)
    p0 = parts[:_N]
    p1 = parts[_N:]

    w0p = jnp.zeros((_F, _F), jnp.float32).at[:, :127].set(W0.T)
    b0p = jnp.zeros((1, _F), jnp.float32).at[0, :127].set(b0)
    w1p = jnp.zeros((_F, _F), jnp.float32).at[:127, :127].set(W1.T)
    b1p = jnp.zeros((1, _F), jnp.float32).at[0, :127].set(b1)
    wcp = jnp.zeros((_F, _F), jnp.float32).at[:127, :_CLS].set(Wc.T)
    bcp = jnp.zeros((1, _F), jnp.float32).at[0, :_CLS].set(bc)

    batch3 = batch.reshape(_NB2, 1, _B2)

    olog, oprob = _tail_call(x_tan, p0, p1, batch3,
                             w0p, b0p, w1p, b1p, wcp, bcp)
    return olog[:, :_CLS], oprob[:, :_CLS]


# speedup vs baseline: 9.1922x; 1.4523x over previous
import functools

import jax
import jax.numpy as jnp
from jax import lax
from jax.experimental import pallas as pl
from jax.experimental.pallas import tpu as pltpu
from jax.experimental.pallas import tpu_sc as plsc

_N = 10000
_E = 320000
_F = 128
_G = 64
_CLS = 39
_EPS = 1e-6
_MAXN = 1000.0

_NC = 2
_NS = 16
_EPT = _E // (_NC * _NS)
_CH = 128
_NFULL = _EPT // _CH
_TAIL = _EPT - _NFULL * _CH
_BCH = 6
_NBLK = _NFULL // _BCH
_BE = _BCH * _CH
_RPT = 624
_RREM = _N - _NS * _RPT


def _sinh(a):
    return 0.5 * (jnp.exp(a) - jnp.exp(-a))


def _logmap_tail(head, tail, sqrt_c):
    z = jnp.clip(head / sqrt_c + _EPS, 1.0, None)
    dist = sqrt_c * jnp.log(z + jnp.sqrt(jnp.clip(z * z - 1.0, 1e-12, None)))
    tmp = jnp.sqrt(jnp.clip(jnp.sum(tail * tail, axis=1, keepdims=True),
                            1e-12, None) + _EPS)
    return (dist / tmp) * tail


def _expmap_norm(tail, sqrt_c):
    sq = jnp.sum(tail * tail, axis=1, keepdims=True)
    lnorm = jnp.sqrt(jnp.clip(sq + _EPS, 1e-6, None))
    cut = jnp.minimum(lnorm, 50.0)
    coef = sqrt_c * _sinh(cut / sqrt_c) / lnorm
    t2 = coef * tail
    n2 = jnp.sum(t2 * t2, axis=1, keepdims=True)
    norm = jnp.sqrt(jnp.clip(n2, 1e-12, None))
    scale = jnp.minimum(1.0, _MAXN / norm)
    t2 = t2 * scale
    head = jnp.sqrt(sqrt_c * sqrt_c
                    + jnp.sum(t2 * t2, axis=1, keepdims=True))
    return head, t2


def _xtan_body(x_ref, o_ref):
    t = x_ref[...]
    n2 = jnp.sum(t * t, axis=1, keepdims=True)
    norm = jnp.sqrt(jnp.clip(n2, 1e-12, None))
    t = t * jnp.minimum(1.0, _MAXN / norm)
    n2s = jnp.sum(t * t, axis=1, keepdims=True)
    head = jnp.sqrt(1.0 + n2s)
    z = jnp.clip(head + _EPS, 1.0, None)
    dist = jnp.log(z + jnp.sqrt(jnp.clip(z * z - 1.0, 1e-12, None)))
    tmp = jnp.sqrt(jnp.clip(n2s, 1e-12, None) + _EPS)
    o_ref[...] = (dist / tmp) * t


def _xtan(xt):
    b = 2000
    return pl.pallas_call(
        _xtan_body,
        grid=(_N // b,),
        in_specs=[pl.BlockSpec((b, _F), lambda i: (i, 0))],
        out_specs=pl.BlockSpec((b, _F), lambda i: (i, 0)),
        out_shape=jax.ShapeDtypeStruct((_N, _F), jnp.float32),
    )(xt)


def _sc_body(xtan_hbm, src_hbm, dst_hbm, zero_hbm, out_hbm,
             sblk, dblk, rows0, rows1, sidx_t, didx_t, rows_t, acc,
             sem, gsem, ssem):
    c = lax.axis_index("c")
    s = lax.axis_index("s")
    tid = c * _NS + s

    rowbase = s * _RPT
    pltpu.sync_copy(zero_hbm, acc.at[pl.ds(rowbase, _RPT)])

    @pl.when(s == _NS - 1)
    def _():
        pltpu.sync_copy(zero_hbm.at[pl.ds(0, _RREM)],
                        acc.at[pl.ds(_NS * _RPT, _RREM)])

    plsc.subcore_barrier()

    ebase = tid * _EPT
    rows = (rows0, rows1)

    def blk_body(blk, carry):
        base = ebase + blk * _BE
        pltpu.sync_copy(src_hbm.at[pl.ds(base, _BE)], sblk)
        pltpu.sync_copy(dst_hbm.at[pl.ds(base, _BE)], dblk)
        g = [None] * _BCH
        s = [None] * _BCH
        g[0] = pltpu.async_copy(xtan_hbm.at[sblk.at[pl.ds(0, _CH)]],
                                rows0, gsem)
        for k in range(1, _BCH):
            cur = rows[(k - 1) % 2]
            nxt = rows[k % 2]
            if k >= 2:
                s[k - 2].wait()
            g[k] = pltpu.async_copy(
                xtan_hbm.at[sblk.at[pl.ds(k * _CH, _CH)]], nxt, gsem)
            g[k - 1].wait()
            s[k - 1] = pltpu.async_copy(
                cur, acc.at[dblk.at[pl.ds((k - 1) * _CH, _CH)]],
                ssem, add=True)
        last = rows[(_BCH - 1) % 2]
        g[_BCH - 1].wait()
        s[_BCH - 1] = pltpu.async_copy(
            last, acc.at[dblk.at[pl.ds((_BCH - 1) * _CH, _CH)]],
            ssem, add=True)
        s[_BCH - 2].wait()
        s[_BCH - 1].wait()
        return carry

    lax.fori_loop(0, _NBLK, blk_body, 0)

    start = ebase + _NFULL * _CH
    pltpu.sync_copy(src_hbm.at[pl.ds(start, _TAIL)], sidx_t)
    pltpu.sync_copy(dst_hbm.at[pl.ds(start, _TAIL)], didx_t)
    pltpu.async_copy(xtan_hbm.at[sidx_t], rows_t, sem).wait()
    pltpu.sync_copy(rows_t, acc.at[didx_t], add=True)

    plsc.subcore_barrier()

    pltpu.sync_copy(acc.at[pl.ds(rowbase, _RPT)],
                    out_hbm.at[pl.ds(c * _N + rowbase, _RPT)])

    @pl.when(s == _NS - 1)
    def _():
        pltpu.sync_copy(acc.at[pl.ds(_NS * _RPT, _RREM)],
                        out_hbm.at[pl.ds(c * _N + _NS * _RPT, _RREM)])


def _sc_scatter(x_tan, src, dst, zero_rows):
    mesh = plsc.VectorSubcoreMesh(core_axis_name="c", subcore_axis_name="s")
    f = pl.kernel(
        _sc_body,
        out_type=jax.ShapeDtypeStruct((_NC * _N, _F), jnp.float32),
        mesh=mesh,
        scratch_types=[
            pltpu.VMEM((_BE,), jnp.int32),
            pltpu.VMEM((_BE,), jnp.int32),
            pltpu.VMEM((_CH, _F), jnp.float32),
            pltpu.VMEM((_CH, _F), jnp.float32),
            pltpu.VMEM((_TAIL,), jnp.int32),
            pltpu.VMEM((_TAIL,), jnp.int32),
            pltpu.VMEM((_TAIL, _F), jnp.float32),
            pltpu.VMEM_SHARED((_N, _F), jnp.float32),
            pltpu.SemaphoreType.DMA,
            pltpu.SemaphoreType.DMA,
            pltpu.SemaphoreType.DMA,
        ],
    )
    return f(x_tan, src, dst, zero_rows)


_B2 = 2000
_NB2 = _N // _B2


def _tail_body(xt_ref, p0_ref, p1_ref, b_ref,
               w0_ref, b0_ref, w1_ref, b1_ref, wc_ref, bc_ref,
               olog_ref, oprob_ref, acc_ref):
    i = pl.program_id(0)

    htan = xt_ref[...] + p0_ref[...] + p1_ref[...]
    head, tail = _expmap_norm(htan, 1.0)

    tt = _logmap_tail(head, tail, 2.0)
    mx = jnp.dot(tt, w0_ref[...], preferred_element_type=jnp.float32) \
        + b0_ref[...]
    head, tail = _expmap_norm(mx, 2.0)
    tt = jax.nn.relu(_logmap_tail(head, tail, 2.0))
    head, tail = _expmap_norm(tt, 2.0)
    tt = _logmap_tail(head, tail, 2.0)
    mx = jnp.dot(tt, w1_ref[...], preferred_element_type=jnp.float32) \
        + b1_ref[...]
    head, tail = _expmap_norm(mx, 2.0)
    tt = jax.nn.relu(_logmap_tail(head, tail, 2.0))
    head, tail = _expmap_norm(tt, 1.0)
    tt = _logmap_tail(head, tail, 1.0)

    bvals = b_ref[...].reshape(1, _B2)
    gid = lax.broadcasted_iota(jnp.int32, (_G, _B2), 0)
    oh = jnp.where(gid == bvals, 1.0, 0.0)
    pp = jnp.dot(oh, tt, preferred_element_type=jnp.float32)

    @pl.when(i == 0)
    def _():
        acc_ref[...] = pp

    @pl.when(i > 0)
    def _():
        acc_ref[...] = acc_ref[...] + pp

    @pl.when(i == _NB2 - 1)
    def _():
        hp = acc_ref[...]
        head, tail = _expmap_norm(hp, 1.0)
        tt = _logmap_tail(head, tail, 1.0)
        mx = jnp.dot(tt, wc_ref[...], preferred_element_type=jnp.float32) \
            + bc_ref[...]
        head, tail = _expmap_norm(mx, 1.0)
        lt = _logmap_tail(head, tail, 1.0)
        olog_ref[...] = lt
        col = lax.broadcasted_iota(jnp.int32, (_G, _F), 1)
        valid = col < _CLS
        m = jnp.maximum(
            jnp.max(jnp.where(valid, lt, -1e30), axis=1, keepdims=True), 0.0)
        e = jnp.where(valid, jnp.exp(lt - m), 0.0)
        denom = jnp.sum(e, axis=1, keepdims=True) + jnp.exp(-m)
        st = e / denom
        _, tail2 = _expmap_norm(st, 1.0)
        oprob_ref[...] = tail2


def _tail_call(x_tan, p0, p1, batch3, w0p, b0p, w1p, b1p, wcp, bcp):
    blk = lambda i: (i, 0)
    fixed = lambda i: (0, 0)
    return pl.pallas_call(
        _tail_body,
        grid=(_NB2,),
        in_specs=[
            pl.BlockSpec((_B2, _F), blk),
            pl.BlockSpec((_B2, _F), blk),
            pl.BlockSpec((_B2, _F), blk),
            pl.BlockSpec((1, 1, _B2), lambda i: (i, 0, 0)),
            pl.BlockSpec((_F, _F), fixed),
            pl.BlockSpec((1, _F), fixed),
            pl.BlockSpec((_F, _F), fixed),
            pl.BlockSpec((1, _F), fixed),
            pl.BlockSpec((_F, _F), fixed),
            pl.BlockSpec((1, _F), fixed),
        ],
        out_specs=[
            pl.BlockSpec((_G, _F), fixed),
            pl.BlockSpec((_G, _F), fixed),
        ],
        out_shape=[
            jax.ShapeDtypeStruct((_G, _F), jnp.float32),
            jax.ShapeDtypeStruct((_G, _F), jnp.float32),
        ],
        scratch_shapes=[pltpu.VMEM((_G, _F), jnp.float32)],
    )(x_tan, p0, p1, batch3, w0p, b0p, w1p, b1p, wcp, bcp)


def kernel(x, edge_index, batch, W0, b0, W1, b1, Wc, bc):
    xt = x[:, 1:]
    x_tan = _xtan(xt)

    src = edge_index[0]
    dst = edge_index[1]
    zero_rows = jnp.zeros((_RPT, _F), jnp.float32)
    parts = _sc_scatter(x_tan, src, dst, zero_rows)
    p0 = parts[:_N]
    p1 = parts[_N:]

    w0p = jnp.zeros((_F, _F), jnp.float32).at[:, :127].set(W0.T)
    b0p = jnp.zeros((1, _F), jnp.float32).at[0, :127].set(b0)
    w1p = jnp.zeros((_F, _F), jnp.float32).at[:127, :127].set(W1.T)
    b1p = jnp.zeros((1, _F), jnp.float32).at[0, :127].set(b1)
    wcp = jnp.zeros((_F, _F), jnp.float32).at[:127, :_CLS].set(Wc.T)
    bcp = jnp.zeros((1, _F), jnp.float32).at[0, :_CLS].set(bc)

    batch3 = batch.reshape(_NB2, 1, _B2)

    olog, oprob = _tail_call(x_tan, p0, p1, batch3,
                             w0p, b0p, w1p, b1p, wcp, bcp)
    return olog[:, :_CLS], oprob[:, :_CLS]
